# Initial kernel scaffold; baseline (speedup 1.0000x reference)
import jax
import jax.numpy as jnp
from jax.experimental import pallas as pl


def kernel(x, edge_index, Wc1, bc1, Wc2, bc2, Wc3, bc3, Wl1, bl1, Wl2, bl2, Wl3, bl3):
    raise NotImplementedError("write your pallas kernel here")


# speedup vs baseline: 9.7610x; 9.7610x over previous
import functools

import jax
import jax.numpy as jnp
from jax import lax
from jax.experimental import pallas as pl
from jax.experimental.pallas import tpu as pltpu
from jax.experimental.pallas import tpu_sc as plsc

N = 10000
E = 320000
D = 128

NC = 2
NS = 16
NW = NC * NS
EPW = E // NW
G = 128
NFULL = EPW // G
REM = EPW - NFULL * G
NP = 10240
RPT = NP // NS
BR = 400
GRID = N // BR

_f32 = jnp.float32


def _sc_mesh():
  return plsc.VectorSubcoreMesh(core_axis_name="c", subcore_axis_name="s")


def _make_sc_deg():
  scratch = [
      pltpu.VMEM((G,), jnp.int32),
      pltpu.VMEM((REM,), jnp.int32),
      pltpu.VMEM((G, D), _f32),
      pltpu.VMEM((REM, D), _f32),
      pltpu.VMEM_SHARED((NP, D), _f32),
      pltpu.SemaphoreType.DMA,
  ]

  @functools.partial(
      pl.kernel,
      out_type=jax.ShapeDtypeStruct((NC, NP, D), _f32),
      mesh=_sc_mesh(),
      scratch_types=scratch,
  )
  def k(dst_hbm, zeros_hbm, ones_hbm, deg_hbm,
        idx, idx_r, ones_v, ones_r, acc, sem):
    c = lax.axis_index("c")
    s = lax.axis_index("s")
    wid = s * NC + c
    r0 = s * RPT

    pltpu.sync_copy(ones_hbm, ones_v)
    pltpu.sync_copy(ones_hbm.at[pl.ds(0, REM), :], ones_r)
    pltpu.sync_copy(zeros_hbm.at[pl.ds(r0, RPT), :],
                    acc.at[pl.ds(r0, RPT), :])
    plsc.subcore_barrier()

    def body(i, _):
      base = wid * EPW + i * G
      pltpu.sync_copy(dst_hbm.at[pl.ds(base, G)], idx)
      pltpu.sync_copy(ones_v, acc.at[idx], add=True)
      return _

    lax.fori_loop(0, NFULL, body, None)
    base = wid * EPW + NFULL * G
    pltpu.sync_copy(dst_hbm.at[pl.ds(base, REM)], idx_r)
    pltpu.sync_copy(ones_r, acc.at[idx_r], add=True)

    plsc.subcore_barrier()
    pltpu.sync_copy(acc.at[pl.ds(r0, RPT), :],
                    deg_hbm.at[c, pl.ds(r0, RPT), :])

  return k


def _make_sc_agg(nb):
  scratch = [
      pltpu.VMEM((G,), jnp.int32),
      pltpu.VMEM((G,), jnp.int32),
      pltpu.VMEM((G, D), _f32),
      pltpu.VMEM((REM,), jnp.int32),
      pltpu.VMEM((REM,), jnp.int32),
      pltpu.VMEM((REM, D), _f32),
      pltpu.VMEM_SHARED((NP, D), _f32),
      pltpu.SemaphoreType.DMA,
  ]

  @functools.partial(
      pl.kernel,
      out_type=jax.ShapeDtypeStruct((NC * nb, NP, D), _f32),
      mesh=_sc_mesh(),
      scratch_types=scratch,
  )
  def k(src_hbm, dst_hbm, zeros_hbm, *rest):
    g_blocks = rest[:nb]
    z_hbm = rest[nb]
    idx_s, idx_d, rows, idx_sr, idx_dr, rows_r, acc, sem = rest[nb + 1:]
    c = lax.axis_index("c")
    s = lax.axis_index("s")
    wid = s * NC + c
    r0 = s * RPT

    for cb in range(nb):
      pltpu.sync_copy(zeros_hbm.at[pl.ds(r0, RPT), :],
                      acc.at[pl.ds(r0, RPT), :])
      plsc.subcore_barrier()

      def body(i, _, cb=cb):
        base = wid * EPW + i * G
        pltpu.sync_copy(src_hbm.at[pl.ds(base, G)], idx_s)
        pltpu.sync_copy(dst_hbm.at[pl.ds(base, G)], idx_d)
        pltpu.async_copy(g_blocks[cb].at[idx_s], rows, sem).wait()
        pltpu.sync_copy(rows, acc.at[idx_d], add=True)
        return _

      lax.fori_loop(0, NFULL, body, None)
      base = wid * EPW + NFULL * G
      pltpu.sync_copy(src_hbm.at[pl.ds(base, REM)], idx_sr)
      pltpu.sync_copy(dst_hbm.at[pl.ds(base, REM)], idx_dr)
      pltpu.async_copy(g_blocks[cb].at[idx_sr], rows_r, sem).wait()
      pltpu.sync_copy(rows_r, acc.at[idx_dr], add=True)

      plsc.subcore_barrier()
      pltpu.sync_copy(acc.at[pl.ds(r0, RPT), :],
                      z_hbm.at[c * nb + cb, pl.ds(r0, RPT), :])

  return k


def _tc_prep(deg16, x):
  def body(deg_ref, x_ref, dinv_ref, g0_ref):
    deg = deg_ref[0, :, 0] + deg_ref[1, :, 0] + 1.0
    dinv = lax.rsqrt(deg)
    db = jnp.broadcast_to(dinv[:, None], (BR, D))
    dinv_ref[...] = db
    g0_ref[...] = db * x_ref[...]

  return pl.pallas_call(
      body,
      grid=(GRID,),
      in_specs=[
          pl.BlockSpec((NC, BR, D), lambda i: (0, i, 0)),
          pl.BlockSpec((BR, D), lambda i: (i, 0)),
      ],
      out_specs=[
          pl.BlockSpec((BR, D), lambda i: (i, 0)),
          pl.BlockSpec((BR, D), lambda i: (i, 0)),
      ],
      out_shape=[
          jax.ShapeDtypeStruct((N, D), _f32),
          jax.ShapeDtypeStruct((N, D), _f32),
      ],
  )(deg16, x)


def _dot_t(a, w):
  return lax.dot_general(a, w, (((1,), (1,)), ((), ())),
                         precision=lax.Precision.HIGHEST,
                         preferred_element_type=_f32)


def _make_tc_mm(nb, dout):
  nbo = dout // D

  def body(*refs):
    z_ref = refs[0]
    g_refs = refs[1:1 + nb]
    dinv_ref, w_ref, b_ref = refs[1 + nb:4 + nb]
    out_refs = refs[4 + nb:]
    dinv = dinv_ref[...]
    z = z_ref[...]
    s = jnp.concatenate(
        [(z[cb] + z[nb + cb] + g_refs[cb][...]) * dinv for cb in range(nb)],
        axis=1)
    h = jax.nn.relu(_dot_t(s, w_ref[...]) + b_ref[...])
    for ob in range(nbo):
      out_refs[ob][...] = h[:, ob * D:(ob + 1) * D] * dinv

  def run(z, gs, dinv_b, w, b):
    din = nb * D
    return pl.pallas_call(
        body,
        grid=(GRID,),
        in_specs=[pl.BlockSpec((NC * nb, BR, D), lambda i: (0, i, 0))]
        + [pl.BlockSpec((BR, D), lambda i: (i, 0)) for _ in range(nb)]
        + [
            pl.BlockSpec((BR, D), lambda i: (i, 0)),
            pl.BlockSpec((dout, din), lambda i: (0, 0)),
            pl.BlockSpec((1, dout), lambda i: (0, 0)),
        ],
        out_specs=[pl.BlockSpec((BR, D), lambda i: (i, 0))
                   for _ in range(nbo)],
        out_shape=[jax.ShapeDtypeStruct((N, D), _f32) for _ in range(nbo)],
    )(z, *gs, dinv_b, w, b)

  return run


def _tc_head(z3, gs, dinv_b, wc3, bc3, wl1, bl1, wl2, bl2, wl3, bl3):
  nb = 4

  def body(z_ref, g0_ref, g1_ref, g2_ref, g3_ref, dinv_ref,
           wc3_ref, bc3_ref, wl1_ref, bl1_ref, wl2_ref, bl2_ref,
           wl3_ref, bl3_ref, out_ref):
    g_refs = (g0_ref, g1_ref, g2_ref, g3_ref)
    dinv = dinv_ref[...]
    z = z_ref[...]
    s = jnp.concatenate(
        [(z[cb] + z[nb + cb] + g_refs[cb][...]) * dinv for cb in range(nb)],
        axis=1)
    h = jax.nn.relu(_dot_t(s, wc3_ref[...]) + bc3_ref[...])
    h = jax.nn.relu(_dot_t(h, wl1_ref[...]) + bl1_ref[...])
    h = jax.nn.relu(_dot_t(h, wl2_ref[...]) + bl2_ref[...])
    out_ref[...] = jax.nn.relu(_dot_t(h, wl3_ref[...]) + bl3_ref[...])

  def wspec(w):
    return pl.BlockSpec(w.shape, lambda i: (0, 0))

  return pl.pallas_call(
      body,
      grid=(GRID,),
      in_specs=[pl.BlockSpec((NC * nb, BR, D), lambda i: (0, i, 0))]
      + [pl.BlockSpec((BR, D), lambda i: (i, 0)) for _ in range(nb)]
      + [pl.BlockSpec((BR, D), lambda i: (i, 0))]
      + [wspec(wc3), wspec(bc3), wspec(wl1), wspec(bl1),
         wspec(wl2), wspec(bl2), wspec(wl3), wspec(bl3)],
      out_specs=pl.BlockSpec((BR, D), lambda i: (i, 0)),
      out_shape=jax.ShapeDtypeStruct((N, D), _f32),
  )(z3, *gs, dinv_b, wc3, bc3, wl1, bl1, wl2, bl2, wl3, bl3)


_sc_deg = _make_sc_deg()
_sc_agg1 = _make_sc_agg(1)
_sc_agg2 = _make_sc_agg(2)
_sc_agg3 = _make_sc_agg(4)
_tc_mm1 = _make_tc_mm(1, 2 * D)
_tc_mm2 = _make_tc_mm(2, 4 * D)


def kernel(x, edge_index, Wc1, bc1, Wc2, bc2, Wc3, bc3,
           Wl1, bl1, Wl2, bl2, Wl3, bl3):
  src = edge_index[0]
  dst = edge_index[1]
  zeros_hbm = jnp.zeros((NP, D), _f32)
  ones128 = jnp.ones((G, D), _f32)

  deg16 = _sc_deg(dst, zeros_hbm, ones128)
  dinv_b, g0 = _tc_prep(deg16, x)

  z1 = _sc_agg1(src, dst, zeros_hbm, g0)
  g1 = _tc_mm1(z1, (g0,), dinv_b, Wc1, bc1.reshape(1, -1))

  z2 = _sc_agg2(src, dst, zeros_hbm, *g1)
  g2 = _tc_mm2(z2, g1, dinv_b, Wc2, bc2.reshape(1, -1))

  z3 = _sc_agg3(src, dst, zeros_hbm, *g2)
  out = _tc_head(z3, g2, dinv_b, Wc3, bc3.reshape(1, -1),
                 Wl1, bl1.reshape(1, -1), Wl2, bl2.reshape(1, -1),
                 Wl3, bl3.reshape(1, -1))
  return out


# speedup vs baseline: 15.4870x; 1.5866x over previous
import functools

import jax
import jax.numpy as jnp
from jax import lax
from jax.experimental import pallas as pl
from jax.experimental.pallas import tpu as pltpu
from jax.experimental.pallas import tpu_sc as plsc

N = 10000
E = 320000
D = 128

NC = 2
NS = 16
NW = NC * NS
G = 128
NROW = E // G
RPW = 80
NRPAD = NW * RPW
NBODY = RPW // 2
NP = 10240
RPT = NP // NS
BR = 400
GRID = N // BR

_f32 = jnp.float32


def _sc_mesh():
  return plsc.VectorSubcoreMesh(core_axis_name="c", subcore_axis_name="s")


def _make_sc_deg():
  scratch = [
      pltpu.VMEM((RPW, G), jnp.int32),
      pltpu.VMEM((G, D), _f32),
      pltpu.VMEM_SHARED((NP, D), _f32),
      pltpu.SemaphoreType.DMA,
  ]

  @functools.partial(
      pl.kernel,
      out_type=jax.ShapeDtypeStruct((NC, NP, D), _f32),
      mesh=_sc_mesh(),
      scratch_types=scratch,
  )
  def k(dst2d_hbm, zeros_hbm, ones_hbm, deg_hbm, dst_rows, ones_v, acc, sem):
    c = lax.axis_index("c")
    s = lax.axis_index("s")
    wid = s * NC + c
    start = RPW * wid
    nrows = jnp.clip(NROW - start, 0, RPW)
    r0 = s * RPT

    pltpu.sync_copy(ones_hbm, ones_v)
    pltpu.sync_copy(dst2d_hbm.at[pl.ds(start, RPW), :], dst_rows)
    pltpu.sync_copy(zeros_hbm.at[pl.ds(r0, RPT), :],
                    acc.at[pl.ds(r0, RPT), :])
    plsc.subcore_barrier()

    def issue(r):
      pltpu.async_copy(ones_v, acc.at[dst_rows.at[r]], sem, add=True)

    def drain_one():
      pltpu.make_async_copy(ones_hbm, ones_v, sem).wait()

    for j in range(4):
      @pl.when(j < nrows)
      def _(j=j):
        issue(j)

    def body(i, _):
      r = i + 4
      @pl.when(r < nrows)
      def _():
        drain_one()
        issue(r)
      return _

    lax.fori_loop(0, RPW - 4, body, None)
    for j in range(4):
      @pl.when(nrows - 4 + j >= 0)
      def _(j=j):
        drain_one()

    plsc.subcore_barrier()
    pltpu.sync_copy(acc.at[pl.ds(r0, RPT), :],
                    deg_hbm.at[c, pl.ds(r0, RPT), :])

  return k


def _make_sc_agg(nb):
  scratch = [
      pltpu.VMEM((RPW, G), jnp.int32),
      pltpu.VMEM((G,), jnp.int32),
      pltpu.VMEM((G,), jnp.int32),
      pltpu.VMEM((G, D), _f32),
      pltpu.VMEM((G, D), _f32),
      pltpu.VMEM_SHARED((NP, D), _f32),
  ] + [pltpu.SemaphoreType.DMA for _ in range(4)]

  @functools.partial(
      pl.kernel,
      out_type=jax.ShapeDtypeStruct((NC * nb, NP, D), _f32),
      mesh=_sc_mesh(),
      scratch_types=scratch,
  )
  def k(src1d_hbm, dst2d_hbm, zeros_hbm, *rest):
    g_blocks = rest[:nb]
    z_hbm = rest[nb]
    dst_rows, i0, i1, r0buf, r1buf, acc, sg0, sg1, ss0, ss1 = rest[nb + 1:]
    idxs = (i0, i1)
    bufs = (r0buf, r1buf)
    gsems = (sg0, sg1)
    ssems = (ss0, ss1)
    c = lax.axis_index("c")
    s = lax.axis_index("s")
    wid = s * NC + c
    start = RPW * wid
    nrows = jnp.clip(NROW - start, 0, RPW)
    rb0 = s * RPT

    pltpu.sync_copy(dst2d_hbm.at[pl.ds(start, RPW), :], dst_rows)

    for cb in range(nb):
      gcb = g_blocks[cb]

      def load_idx(j, r):
        pltpu.sync_copy(src1d_hbm.at[pl.ds((start + r) * G, G)], idxs[j])

      def issue_gather(j, gcb=gcb):
        pltpu.async_copy(gcb.at[idxs[j]], bufs[j], gsems[j])

      def wait_gather(j, gcb=gcb):
        pltpu.make_async_copy(gcb.at[pl.ds(0, G)], bufs[j], gsems[j]).wait()

      def issue_scatter(j, r):
        pltpu.async_copy(bufs[j], acc.at[dst_rows.at[r]], ssems[j], add=True)

      def wait_scatter(j):
        pltpu.make_async_copy(bufs[j], acc.at[pl.ds(0, G)], ssems[j]).wait()

      pltpu.sync_copy(zeros_hbm.at[pl.ds(rb0, RPT), :],
                      acc.at[pl.ds(rb0, RPT), :])
      plsc.subcore_barrier()

      for j in range(2):
        @pl.when(j < nrows)
        def _(j=j):
          load_idx(j, j)
          issue_gather(j)

      def body(i, _):
        for j in range(2):
          r = 2 * i + j
          @pl.when(r < nrows)
          def _(j=j, r=r):
            wait_gather(j)
            issue_scatter(j, r)
        for j in range(2):
          r = 2 * i + j
          rn = r + 2
          @pl.when(r < nrows)
          def _(j=j, r=r):
            wait_scatter(j)
          @pl.when(rn < nrows)
          def _(j=j, rn=rn):
            load_idx(j, rn)
            issue_gather(j)
        return _

      lax.fori_loop(0, NBODY, body, None)

      plsc.subcore_barrier()
      pltpu.sync_copy(acc.at[pl.ds(rb0, RPT), :],
                      z_hbm.at[c * nb + cb, pl.ds(rb0, RPT), :])

  return k


def _tc_prep(deg16, x):
  def body(deg_ref, x_ref, dinv_ref, g0_ref):
    deg = deg_ref[0, :, 0] + deg_ref[1, :, 0] + 1.0
    dinv = lax.rsqrt(deg)
    db = jnp.broadcast_to(dinv[:, None], (BR, D))
    dinv_ref[...] = db
    g0_ref[...] = db * x_ref[...]

  return pl.pallas_call(
      body,
      grid=(GRID,),
      in_specs=[
          pl.BlockSpec((NC, BR, D), lambda i: (0, i, 0)),
          pl.BlockSpec((BR, D), lambda i: (i, 0)),
      ],
      out_specs=[
          pl.BlockSpec((BR, D), lambda i: (i, 0)),
          pl.BlockSpec((BR, D), lambda i: (i, 0)),
      ],
      out_shape=[
          jax.ShapeDtypeStruct((N, D), _f32),
          jax.ShapeDtypeStruct((N, D), _f32),
      ],
  )(deg16, x)


def _dot_t(a, w):
  return lax.dot_general(a, w, (((1,), (1,)), ((), ())),
                         precision=lax.Precision.HIGHEST,
                         preferred_element_type=_f32)


def _make_tc_mm(nb, dout):
  nbo = dout // D

  def body(*refs):
    z_ref = refs[0]
    g_refs = refs[1:1 + nb]
    dinv_ref, w_ref, b_ref = refs[1 + nb:4 + nb]
    out_refs = refs[4 + nb:]
    dinv = dinv_ref[...]
    z = z_ref[...]
    s = jnp.concatenate(
        [(z[cb] + z[nb + cb] + g_refs[cb][...]) * dinv for cb in range(nb)],
        axis=1)
    h = jax.nn.relu(_dot_t(s, w_ref[...]) + b_ref[...])
    for ob in range(nbo):
      out_refs[ob][...] = h[:, ob * D:(ob + 1) * D] * dinv

  def run(z, gs, dinv_b, w, b):
    din = nb * D
    return pl.pallas_call(
        body,
        grid=(GRID,),
        in_specs=[pl.BlockSpec((NC * nb, BR, D), lambda i: (0, i, 0))]
        + [pl.BlockSpec((BR, D), lambda i: (i, 0)) for _ in range(nb)]
        + [
            pl.BlockSpec((BR, D), lambda i: (i, 0)),
            pl.BlockSpec((dout, din), lambda i: (0, 0)),
            pl.BlockSpec((1, dout), lambda i: (0, 0)),
        ],
        out_specs=[pl.BlockSpec((BR, D), lambda i: (i, 0))
                   for _ in range(nbo)],
        out_shape=[jax.ShapeDtypeStruct((N, D), _f32) for _ in range(nbo)],
    )(z, *gs, dinv_b, w, b)

  return run


def _tc_head(z3, gs, dinv_b, wc3, bc3, wl1, bl1, wl2, bl2, wl3, bl3):
  nb = 4

  def body(z_ref, g0_ref, g1_ref, g2_ref, g3_ref, dinv_ref,
           wc3_ref, bc3_ref, wl1_ref, bl1_ref, wl2_ref, bl2_ref,
           wl3_ref, bl3_ref, out_ref):
    g_refs = (g0_ref, g1_ref, g2_ref, g3_ref)
    dinv = dinv_ref[...]
    z = z_ref[...]
    s = jnp.concatenate(
        [(z[cb] + z[nb + cb] + g_refs[cb][...]) * dinv for cb in range(nb)],
        axis=1)
    h = jax.nn.relu(_dot_t(s, wc3_ref[...]) + bc3_ref[...])
    h = jax.nn.relu(_dot_t(h, wl1_ref[...]) + bl1_ref[...])
    h = jax.nn.relu(_dot_t(h, wl2_ref[...]) + bl2_ref[...])
    out_ref[...] = jax.nn.relu(_dot_t(h, wl3_ref[...]) + bl3_ref[...])

  def wspec(w):
    return pl.BlockSpec(w.shape, lambda i: (0, 0))

  return pl.pallas_call(
      body,
      grid=(GRID,),
      in_specs=[pl.BlockSpec((NC * nb, BR, D), lambda i: (0, i, 0))]
      + [pl.BlockSpec((BR, D), lambda i: (i, 0)) for _ in range(nb)]
      + [pl.BlockSpec((BR, D), lambda i: (i, 0))]
      + [wspec(wc3), wspec(bc3), wspec(wl1), wspec(bl1),
         wspec(wl2), wspec(bl2), wspec(wl3), wspec(bl3)],
      out_specs=pl.BlockSpec((BR, D), lambda i: (i, 0)),
      out_shape=jax.ShapeDtypeStruct((N, D), _f32),
  )(z3, *gs, dinv_b, wc3, bc3, wl1, bl1, wl2, bl2, wl3, bl3)


_sc_deg = _make_sc_deg()
_sc_agg1 = _make_sc_agg(1)
_sc_agg2 = _make_sc_agg(2)
_sc_agg3 = _make_sc_agg(4)
_tc_mm1 = _make_tc_mm(1, 2 * D)
_tc_mm2 = _make_tc_mm(2, 4 * D)


def kernel(x, edge_index, Wc1, bc1, Wc2, bc2, Wc3, bc3,
           Wl1, bl1, Wl2, bl2, Wl3, bl3):
  src1d = edge_index[0]
  dst2d = jnp.concatenate(
      [edge_index[1], jnp.zeros((NRPAD * G - E,), jnp.int32)]).reshape(NRPAD, G)
  zeros_hbm = jnp.zeros((NP, D), _f32)
  ones128 = jnp.ones((G, D), _f32)

  deg16 = _sc_deg(dst2d, zeros_hbm, ones128)
  dinv_b, g0 = _tc_prep(deg16, x)

  z1 = _sc_agg1(src1d, dst2d, zeros_hbm, g0)
  g1 = _tc_mm1(z1, (g0,), dinv_b, Wc1, bc1.reshape(1, -1))

  z2 = _sc_agg2(src1d, dst2d, zeros_hbm, *g1)
  g2 = _tc_mm2(z2, g1, dinv_b, Wc2, bc2.reshape(1, -1))

  z3 = _sc_agg3(src1d, dst2d, zeros_hbm, *g2)
  out = _tc_head(z3, g2, dinv_b, Wc3, bc3.reshape(1, -1),
                 Wl1, bl1.reshape(1, -1), Wl2, bl2.reshape(1, -1),
                 Wl3, bl3.reshape(1, -1))
  return out


# speedup vs baseline: 17.4777x; 1.1285x over previous
import functools

import jax
import jax.numpy as jnp
from jax import lax
from jax.experimental import pallas as pl
from jax.experimental.pallas import tpu as pltpu
from jax.experimental.pallas import tpu_sc as plsc

N = 10000
E = 320000
D = 128

NC = 2
NS = 16
NW = NC * NS
G = 128
NROW = E // G
RPW = 80
NRPAD = NW * RPW
NBODY = RPW // 2
NP = 10240
RPT = NP // NS
BR = 400
GRID = N // BR

_f32 = jnp.float32


def _sc_mesh():
  return plsc.VectorSubcoreMesh(core_axis_name="c", subcore_axis_name="s")


def _make_sc_deg():
  scratch = [
      pltpu.VMEM((RPW, G), jnp.int32),
      pltpu.VMEM((G, D), _f32),
      pltpu.VMEM_SHARED((NP, D), _f32),
      pltpu.SemaphoreType.DMA,
  ]

  @functools.partial(
      pl.kernel,
      out_type=jax.ShapeDtypeStruct((NC, NP, D), _f32),
      mesh=_sc_mesh(),
      scratch_types=scratch,
  )
  def k(dst2d_hbm, zeros_hbm, ones_hbm, deg_hbm, dst_rows, ones_v, acc, sem):
    c = lax.axis_index("c")
    s = lax.axis_index("s")
    wid = s * NC + c
    start = RPW * wid
    nrows = jnp.clip(NROW - start, 0, RPW)
    r0 = s * RPT

    pltpu.sync_copy(ones_hbm, ones_v)
    pltpu.sync_copy(dst2d_hbm.at[pl.ds(start, RPW), :], dst_rows)
    pltpu.sync_copy(zeros_hbm.at[pl.ds(r0, RPT), :],
                    acc.at[pl.ds(r0, RPT), :])
    plsc.subcore_barrier()

    def issue(r):
      pltpu.async_copy(ones_v, acc.at[dst_rows.at[r]], sem, add=True)

    def drain_one():
      pltpu.make_async_copy(ones_hbm, ones_v, sem).wait()

    for j in range(4):
      @pl.when(j < nrows)
      def _(j=j):
        issue(j)

    def body(i, _):
      r = i + 4
      @pl.when(r < nrows)
      def _():
        drain_one()
        issue(r)
      return _

    lax.fori_loop(0, RPW - 4, body, None)
    for j in range(4):
      @pl.when(nrows - 4 + j >= 0)
      def _(j=j):
        drain_one()

    plsc.subcore_barrier()
    pltpu.sync_copy(acc.at[pl.ds(r0, RPT), :],
                    deg_hbm.at[c, pl.ds(r0, RPT), :])

  return k


def _make_sc_agg(nb):
  scratch = [
      pltpu.VMEM((RPW, G), jnp.int32),
      pltpu.VMEM((G,), jnp.int32),
      pltpu.VMEM((G,), jnp.int32),
      pltpu.VMEM((G, D), _f32),
      pltpu.VMEM((G, D), _f32),
      pltpu.VMEM_SHARED((NP, D), _f32),
  ] + [pltpu.SemaphoreType.DMA for _ in range(4)]

  @functools.partial(
      pl.kernel,
      out_type=jax.ShapeDtypeStruct((NC * nb, NP, D), _f32),
      mesh=_sc_mesh(),
      scratch_types=scratch,
  )
  def k(src1d_hbm, dst2d_hbm, zeros_hbm, *rest):
    g_blocks = rest[:nb]
    z_hbm = rest[nb]
    dst_rows, i0, i1, r0buf, r1buf, acc, sg0, sg1, ss0, ss1 = rest[nb + 1:]
    idxs = (i0, i1)
    bufs = (r0buf, r1buf)
    gsems = (sg0, sg1)
    ssems = (ss0, ss1)
    c = lax.axis_index("c")
    s = lax.axis_index("s")
    wid = s * NC + c
    start = RPW * wid
    nrows = jnp.clip(NROW - start, 0, RPW)
    rb0 = s * RPT

    pltpu.sync_copy(dst2d_hbm.at[pl.ds(start, RPW), :], dst_rows)

    for cb in range(nb):
      gcb = g_blocks[cb]

      def load_idx(j, r):
        pltpu.sync_copy(src1d_hbm.at[pl.ds((start + r) * G, G)], idxs[j])

      def issue_gather(j, gcb=gcb):
        pltpu.async_copy(gcb.at[idxs[j]], bufs[j], gsems[j])

      def wait_gather(j, gcb=gcb):
        pltpu.make_async_copy(gcb.at[pl.ds(0, G)], bufs[j], gsems[j]).wait()

      def issue_scatter(j, r):
        pltpu.async_copy(bufs[j], acc.at[dst_rows.at[r]], ssems[j], add=True)

      def wait_scatter(j):
        pltpu.make_async_copy(bufs[j], acc.at[pl.ds(0, G)], ssems[j]).wait()

      pltpu.sync_copy(zeros_hbm.at[pl.ds(rb0, RPT), :],
                      acc.at[pl.ds(rb0, RPT), :])
      plsc.subcore_barrier()

      for j in range(2):
        @pl.when(j < nrows)
        def _(j=j):
          load_idx(j, j)
          issue_gather(j)

      def body(i, _):
        for j in range(2):
          r = 2 * i + j
          @pl.when(r < nrows)
          def _(j=j, r=r):
            wait_gather(j)
            issue_scatter(j, r)
        for j in range(2):
          r = 2 * i + j
          rn = r + 2
          @pl.when(r < nrows)
          def _(j=j, r=r):
            wait_scatter(j)
          @pl.when(rn < nrows)
          def _(j=j, rn=rn):
            load_idx(j, rn)
            issue_gather(j)
        return _

      lax.fori_loop(0, NBODY, body, None)

      plsc.subcore_barrier()
      pltpu.sync_copy(acc.at[pl.ds(rb0, RPT), :],
                      z_hbm.at[c * nb + cb, pl.ds(rb0, RPT), :])

  return k


def _tc_prep(deg16, x):
  def body(deg_ref, x_ref, dinv_ref, g0_ref):
    deg = deg_ref[0, :, 0] + deg_ref[1, :, 0] + 1.0
    dinv = 1.0 / lax.sqrt(deg)
    db = jnp.broadcast_to(dinv[:, None], (BR, D))
    dinv_ref[...] = db
    g0_ref[...] = db * x_ref[...]

  return pl.pallas_call(
      body,
      grid=(GRID,),
      in_specs=[
          pl.BlockSpec((NC, BR, D), lambda i: (0, i, 0)),
          pl.BlockSpec((BR, D), lambda i: (i, 0)),
      ],
      out_specs=[
          pl.BlockSpec((BR, D), lambda i: (i, 0)),
          pl.BlockSpec((BR, D), lambda i: (i, 0)),
      ],
      out_shape=[
          jax.ShapeDtypeStruct((N, D), _f32),
          jax.ShapeDtypeStruct((N, D), _f32),
      ],
  )(deg16, x)


def _dot_t(a, w):
  return lax.dot_general(a, w, (((1,), (1,)), ((), ())),
                         precision=lax.Precision.DEFAULT,
                         preferred_element_type=_f32)


def _make_tc_mm(nb, dout):
  nbo = dout // D

  def body(*refs):
    z_ref = refs[0]
    g_refs = refs[1:1 + nb]
    dinv_ref, w_ref, b_ref = refs[1 + nb:4 + nb]
    out_refs = refs[4 + nb:]
    dinv = dinv_ref[...]
    z = z_ref[...]
    s = jnp.concatenate(
        [(z[cb] + z[nb + cb] + g_refs[cb][...]) * dinv for cb in range(nb)],
        axis=1)
    h = jax.nn.relu(_dot_t(s, w_ref[...]) + b_ref[...])
    for ob in range(nbo):
      out_refs[ob][...] = h[:, ob * D:(ob + 1) * D] * dinv

  def run(z, gs, dinv_b, w, b):
    din = nb * D
    return pl.pallas_call(
        body,
        grid=(GRID,),
        in_specs=[pl.BlockSpec((NC * nb, BR, D), lambda i: (0, i, 0))]
        + [pl.BlockSpec((BR, D), lambda i: (i, 0)) for _ in range(nb)]
        + [
            pl.BlockSpec((BR, D), lambda i: (i, 0)),
            pl.BlockSpec((dout, din), lambda i: (0, 0)),
            pl.BlockSpec((1, dout), lambda i: (0, 0)),
        ],
        out_specs=[pl.BlockSpec((BR, D), lambda i: (i, 0))
                   for _ in range(nbo)],
        out_shape=[jax.ShapeDtypeStruct((N, D), _f32) for _ in range(nbo)],
    )(z, *gs, dinv_b, w, b)

  return run


def _tc_head(z3, gs, dinv_b, wc3, bc3, wl1, bl1, wl2, bl2, wl3, bl3):
  nb = 4

  def body(z_ref, g0_ref, g1_ref, g2_ref, g3_ref, dinv_ref,
           wc3_ref, bc3_ref, wl1_ref, bl1_ref, wl2_ref, bl2_ref,
           wl3_ref, bl3_ref, out_ref):
    g_refs = (g0_ref, g1_ref, g2_ref, g3_ref)
    dinv = dinv_ref[...]
    z = z_ref[...]
    s = jnp.concatenate(
        [(z[cb] + z[nb + cb] + g_refs[cb][...]) * dinv for cb in range(nb)],
        axis=1)
    h = jax.nn.relu(_dot_t(s, wc3_ref[...]) + bc3_ref[...])
    h = jax.nn.relu(_dot_t(h, wl1_ref[...]) + bl1_ref[...])
    h = jax.nn.relu(_dot_t(h, wl2_ref[...]) + bl2_ref[...])
    out_ref[...] = jax.nn.relu(_dot_t(h, wl3_ref[...]) + bl3_ref[...])

  def wspec(w):
    return pl.BlockSpec(w.shape, lambda i: (0, 0))

  return pl.pallas_call(
      body,
      grid=(GRID,),
      in_specs=[pl.BlockSpec((NC * nb, BR, D), lambda i: (0, i, 0))]
      + [pl.BlockSpec((BR, D), lambda i: (i, 0)) for _ in range(nb)]
      + [pl.BlockSpec((BR, D), lambda i: (i, 0))]
      + [wspec(wc3), wspec(bc3), wspec(wl1), wspec(bl1),
         wspec(wl2), wspec(bl2), wspec(wl3), wspec(bl3)],
      out_specs=pl.BlockSpec((BR, D), lambda i: (i, 0)),
      out_shape=jax.ShapeDtypeStruct((N, D), _f32),
  )(z3, *gs, dinv_b, wc3, bc3, wl1, bl1, wl2, bl2, wl3, bl3)


_sc_deg = _make_sc_deg()
_sc_agg1 = _make_sc_agg(1)
_sc_agg2 = _make_sc_agg(2)
_sc_agg3 = _make_sc_agg(4)
_tc_mm1 = _make_tc_mm(1, 2 * D)
_tc_mm2 = _make_tc_mm(2, 4 * D)


def kernel(x, edge_index, Wc1, bc1, Wc2, bc2, Wc3, bc3,
           Wl1, bl1, Wl2, bl2, Wl3, bl3):
  src1d = edge_index[0]
  dst2d = jnp.concatenate(
      [edge_index[1], jnp.zeros((NRPAD * G - E,), jnp.int32)]).reshape(NRPAD, G)
  zeros_hbm = jnp.zeros((NP, D), _f32)

  ones128 = jnp.ones((G, D), _f32)
  deg16 = _sc_deg(dst2d, zeros_hbm, ones128)
  dinv_b, g0 = _tc_prep(deg16, x)

  z1 = _sc_agg1(src1d, dst2d, zeros_hbm, g0)
  g1 = _tc_mm1(z1, (g0,), dinv_b, Wc1, bc1.reshape(1, -1))

  z2 = _sc_agg2(src1d, dst2d, zeros_hbm, *g1)
  g2 = _tc_mm2(z2, g1, dinv_b, Wc2, bc2.reshape(1, -1))

  z3 = _sc_agg3(src1d, dst2d, zeros_hbm, *g2)
  out = _tc_head(z3, g2, dinv_b, Wc3, bc3.reshape(1, -1),
                 Wl1, bl1.reshape(1, -1), Wl2, bl2.reshape(1, -1),
                 Wl3, bl3.reshape(1, -1))
  return out


# speedup vs baseline: 18.1374x; 1.0377x over previous
import functools

import jax
import jax.numpy as jnp
from jax import lax
from jax.experimental import pallas as pl
from jax.experimental.pallas import tpu as pltpu
from jax.experimental.pallas import tpu_sc as plsc

N = 10000
E = 320000
D = 128

NC = 2
NS = 16
NW = NC * NS
G = 128
NROW = E // G
RPW = 80
NP = 10112
RPT = NP // NS
BR = 400
GRID = N // BR

_f32 = jnp.float32


def _sc_mesh():
  return plsc.VectorSubcoreMesh(core_axis_name="c", subcore_axis_name="s")


def _make_sc_deg():
  scratch = [
      pltpu.VMEM((G, D), _f32),
  ] + [pltpu.VMEM((G,), jnp.int32) for _ in range(4)] + [
      pltpu.VMEM_SHARED((NP, D), _f32),
  ] + [pltpu.SemaphoreType.DMA for _ in range(4)]

  @functools.partial(
      pl.kernel,
      out_type=jax.ShapeDtypeStruct((NC, NP, D), _f32),
      mesh=_sc_mesh(),
      scratch_types=scratch,
  )
  def k(dst_hbm, zeros_hbm, ones_hbm, deg_hbm, ones_v,
        i0, i1, i2, i3, acc, s0, s1, s2, s3):
    idxs = (i0, i1, i2, i3)
    sems = (s0, s1, s2, s3)
    c = lax.axis_index("c")
    s = lax.axis_index("s")
    wid = s * NC + c
    start = RPW * wid
    nrows = jnp.clip(NROW - start, 0, RPW)
    r0 = s * RPT

    pltpu.sync_copy(ones_hbm, ones_v)
    pltpu.sync_copy(zeros_hbm.at[pl.ds(r0, RPT), :],
                    acc.at[pl.ds(r0, RPT), :])
    plsc.subcore_barrier()

    def fire(j, r):
      pltpu.sync_copy(dst_hbm.at[pl.ds((start + r) * G, G)], idxs[j])
      pltpu.async_copy(ones_v, acc.at[idxs[j]], sems[j], add=True)

    def drain(j):
      pltpu.make_async_copy(ones_hbm, ones_v, sems[j]).wait()

    for j in range(4):
      @pl.when(j < nrows)
      def _(j=j):
        fire(j, j)

    def body(i, _):
      for j in range(4):
        r = 4 + 4 * i + j
        @pl.when(r < nrows)
        def _(j=j, r=r):
          drain(j)
          fire(j, r)
      return _

    lax.fori_loop(0, (RPW - 4) // 4, body, None)
    for j in range(4):
      @pl.when(j < nrows)
      def _(j=j):
        drain(j)

    plsc.subcore_barrier()
    pltpu.sync_copy(acc.at[pl.ds(r0, RPT), :],
                    deg_hbm.at[c, pl.ds(r0, RPT), :])

  return k


NRING = 3
NBODY = -(-RPW // NRING)


def _make_sc_agg(nb):
  scratch = (
      [pltpu.VMEM((G,), jnp.int32) for _ in range(NRING)]
      + [pltpu.VMEM((G,), jnp.int32) for _ in range(NRING)]
      + [pltpu.VMEM((G, D), _f32) for _ in range(NRING)]
      + [pltpu.VMEM_SHARED((NP, D), _f32)]
      + [pltpu.SemaphoreType.DMA for _ in range(2 * NRING)]
  )

  @functools.partial(
      pl.kernel,
      out_type=jax.ShapeDtypeStruct((NC * nb, NP, D), _f32),
      mesh=_sc_mesh(),
      scratch_types=scratch,
  )
  def k(src_hbm, dst_hbm, zeros_hbm, *rest):
    g_blocks = rest[:nb]
    z_hbm = rest[nb]
    rest = rest[nb + 1:]
    sidx = rest[0:NRING]
    didx = rest[NRING:2 * NRING]
    bufs = rest[2 * NRING:3 * NRING]
    acc = rest[3 * NRING]
    gsems = rest[3 * NRING + 1:3 * NRING + 1 + NRING]
    ssems = rest[3 * NRING + 1 + NRING:]
    c = lax.axis_index("c")
    s = lax.axis_index("s")
    wid = s * NC + c
    start = RPW * wid
    nrows = jnp.clip(NROW - start, 0, RPW)
    rb0 = s * RPT

    for cb in range(nb):
      gcb = g_blocks[cb]

      def load_sidx(j, r):
        pltpu.sync_copy(src_hbm.at[pl.ds((start + r) * G, G)], sidx[j])

      def load_didx(j, r):
        pltpu.sync_copy(dst_hbm.at[pl.ds((start + r) * G, G)], didx[j])

      def issue_gather(j, gcb=gcb):
        pltpu.async_copy(gcb.at[sidx[j]], bufs[j], gsems[j])

      def wait_gather(j, gcb=gcb):
        pltpu.make_async_copy(gcb.at[pl.ds(0, G)], bufs[j], gsems[j]).wait()

      def issue_scatter(j):
        pltpu.async_copy(bufs[j], acc.at[didx[j]], ssems[j], add=True)

      def wait_scatter(j):
        pltpu.make_async_copy(bufs[j], acc.at[pl.ds(0, G)], ssems[j]).wait()

      pltpu.sync_copy(zeros_hbm.at[pl.ds(rb0, RPT), :],
                      acc.at[pl.ds(rb0, RPT), :])
      plsc.subcore_barrier()

      for j in range(NRING):
        @pl.when(j < nrows)
        def _(j=j):
          load_sidx(j, j)
          load_didx(j, j)
          issue_gather(j)

      def body(i, _):
        for j in range(NRING):
          r = NRING * i + j
          rn = r + NRING
          @pl.when(r < nrows)
          def _(j=j):
            wait_gather(j)
            issue_scatter(j)
          @pl.when(rn < nrows)
          def _(j=j, rn=rn):
            load_sidx(j, rn)
        for j in range(NRING):
          r = NRING * i + j
          rn = r + NRING
          @pl.when(r < nrows)
          def _(j=j):
            wait_scatter(j)
          @pl.when(rn < nrows)
          def _(j=j, rn=rn):
            load_didx(j, rn)
            issue_gather(j)
        return _

      lax.fori_loop(0, NBODY, body, None)

      plsc.subcore_barrier()
      pltpu.sync_copy(acc.at[pl.ds(rb0, RPT), :],
                      z_hbm.at[c * nb + cb, pl.ds(rb0, RPT), :])

  return k


def _tc_prep(deg16, x):
  def body(deg_ref, x_ref, dinv_ref, g0_ref):
    deg = deg_ref[0, :, 0] + deg_ref[1, :, 0] + 1.0
    dinv = 1.0 / lax.sqrt(deg)
    db = jnp.broadcast_to(dinv[:, None], (BR, D))
    dinv_ref[...] = db
    g0_ref[...] = db * x_ref[...]

  return pl.pallas_call(
      body,
      grid=(GRID,),
      in_specs=[
          pl.BlockSpec((NC, BR, D), lambda i: (0, i, 0)),
          pl.BlockSpec((BR, D), lambda i: (i, 0)),
      ],
      out_specs=[
          pl.BlockSpec((BR, D), lambda i: (i, 0)),
          pl.BlockSpec((BR, D), lambda i: (i, 0)),
      ],
      out_shape=[
          jax.ShapeDtypeStruct((N, D), _f32),
          jax.ShapeDtypeStruct((N, D), _f32),
      ],
  )(deg16, x)


def _dot_t(a, w):
  return lax.dot_general(a, w, (((1,), (1,)), ((), ())),
                         precision=lax.Precision.DEFAULT,
                         preferred_element_type=_f32)


def _make_tc_mm(nb, dout):
  nbo = dout // D

  def body(*refs):
    z_ref = refs[0]
    g_refs = refs[1:1 + nb]
    dinv_ref, w_ref, b_ref = refs[1 + nb:4 + nb]
    out_refs = refs[4 + nb:]
    dinv = dinv_ref[...]
    z = z_ref[...]
    s = jnp.concatenate(
        [(z[cb] + z[nb + cb] + g_refs[cb][...]) * dinv for cb in range(nb)],
        axis=1)
    h = jax.nn.relu(_dot_t(s, w_ref[...]) + b_ref[...])
    for ob in range(nbo):
      out_refs[ob][...] = h[:, ob * D:(ob + 1) * D] * dinv

  def run(z, gs, dinv_b, w, b):
    din = nb * D
    return pl.pallas_call(
        body,
        grid=(GRID,),
        in_specs=[pl.BlockSpec((NC * nb, BR, D), lambda i: (0, i, 0))]
        + [pl.BlockSpec((BR, D), lambda i: (i, 0)) for _ in range(nb)]
        + [
            pl.BlockSpec((BR, D), lambda i: (i, 0)),
            pl.BlockSpec((dout, din), lambda i: (0, 0)),
            pl.BlockSpec((1, dout), lambda i: (0, 0)),
        ],
        out_specs=[pl.BlockSpec((BR, D), lambda i: (i, 0))
                   for _ in range(nbo)],
        out_shape=[jax.ShapeDtypeStruct((N, D), _f32) for _ in range(nbo)],
    )(z, *gs, dinv_b, w, b)

  return run


def _tc_head(z3, gs, dinv_b, wc3, bc3, wl1, bl1, wl2, bl2, wl3, bl3):
  nb = 4

  def body(z_ref, g0_ref, g1_ref, g2_ref, g3_ref, dinv_ref,
           wc3_ref, bc3_ref, wl1_ref, bl1_ref, wl2_ref, bl2_ref,
           wl3_ref, bl3_ref, out_ref):
    g_refs = (g0_ref, g1_ref, g2_ref, g3_ref)
    dinv = dinv_ref[...]
    z = z_ref[...]
    s = jnp.concatenate(
        [(z[cb] + z[nb + cb] + g_refs[cb][...]) * dinv for cb in range(nb)],
        axis=1)
    h = jax.nn.relu(_dot_t(s, wc3_ref[...]) + bc3_ref[...])
    h = jax.nn.relu(_dot_t(h, wl1_ref[...]) + bl1_ref[...])
    h = jax.nn.relu(_dot_t(h, wl2_ref[...]) + bl2_ref[...])
    out_ref[...] = jax.nn.relu(_dot_t(h, wl3_ref[...]) + bl3_ref[...])

  def wspec(w):
    return pl.BlockSpec(w.shape, lambda i: (0, 0))

  return pl.pallas_call(
      body,
      grid=(GRID,),
      in_specs=[pl.BlockSpec((NC * nb, BR, D), lambda i: (0, i, 0))]
      + [pl.BlockSpec((BR, D), lambda i: (i, 0)) for _ in range(nb)]
      + [pl.BlockSpec((BR, D), lambda i: (i, 0))]
      + [wspec(wc3), wspec(bc3), wspec(wl1), wspec(bl1),
         wspec(wl2), wspec(bl2), wspec(wl3), wspec(bl3)],
      out_specs=pl.BlockSpec((BR, D), lambda i: (i, 0)),
      out_shape=jax.ShapeDtypeStruct((N, D), _f32),
  )(z3, *gs, dinv_b, wc3, bc3, wl1, bl1, wl2, bl2, wl3, bl3)


_sc_deg = _make_sc_deg()
_sc_agg1 = _make_sc_agg(1)
_sc_agg2 = _make_sc_agg(2)
_sc_agg3 = _make_sc_agg(4)
_tc_mm1 = _make_tc_mm(1, 2 * D)
_tc_mm2 = _make_tc_mm(2, 4 * D)


def kernel(x, edge_index, Wc1, bc1, Wc2, bc2, Wc3, bc3,
           Wl1, bl1, Wl2, bl2, Wl3, bl3):
  src1d = edge_index[0]
  dst1d = edge_index[1]
  zeros_hbm = jnp.zeros((NP, D), _f32)
  ones128 = jnp.ones((G, D), _f32)

  deg16 = _sc_deg(dst1d, zeros_hbm, ones128)
  dinv_b, g0 = _tc_prep(deg16, x)

  z1 = _sc_agg1(src1d, dst1d, zeros_hbm, g0)
  g1 = _tc_mm1(z1, (g0,), dinv_b, Wc1, bc1.reshape(1, -1))

  z2 = _sc_agg2(src1d, dst1d, zeros_hbm, *g1)
  g2 = _tc_mm2(z2, g1, dinv_b, Wc2, bc2.reshape(1, -1))

  z3 = _sc_agg3(src1d, dst1d, zeros_hbm, *g2)
  out = _tc_head(z3, g2, dinv_b, Wc3, bc3.reshape(1, -1),
                 Wl1, bl1.reshape(1, -1), Wl2, bl2.reshape(1, -1),
                 Wl3, bl3.reshape(1, -1))
  return out


# speedup vs baseline: 20.2047x; 1.1140x over previous
import functools

import jax
import jax.numpy as jnp
from jax import lax
from jax.experimental import pallas as pl
from jax.experimental.pallas import tpu as pltpu
from jax.experimental.pallas import tpu_sc as plsc

N = 10000
E = 320000
D = 128

NC = 2
NS = 16
NW = NC * NS
G = 128
NROW = E // G
RPW = 80
NROWP = 2504
NP = 10112
RPT = NP // NS
BR = 400
GRID = N // BR

_f32 = jnp.float32


def _sc_mesh():
  return plsc.VectorSubcoreMesh(core_axis_name="c", subcore_axis_name="s")


def _make_sc_deg():
  scratch = [
      pltpu.VMEM((G, D), _f32),
  ] + [pltpu.VMEM((G,), jnp.int32) for _ in range(4)] + [
      pltpu.VMEM_SHARED((NP, D), _f32),
  ] + [pltpu.SemaphoreType.DMA for _ in range(4)]

  @functools.partial(
      pl.kernel,
      out_type=jax.ShapeDtypeStruct((NC, NP, D), _f32),
      mesh=_sc_mesh(),
      scratch_types=scratch,
  )
  def k(dst_hbm, zeros_hbm, ones_hbm, deg_hbm, ones_v,
        i0, i1, i2, i3, acc, s0, s1, s2, s3):
    idxs = (i0, i1, i2, i3)
    sems = (s0, s1, s2, s3)
    c = lax.axis_index("c")
    s = lax.axis_index("s")
    wid = s * NC + c
    start = RPW * wid
    nrows = jnp.clip(NROW - start, 0, RPW)
    r0 = s * RPT

    pltpu.sync_copy(ones_hbm, ones_v)
    pltpu.sync_copy(zeros_hbm.at[pl.ds(r0, RPT), :],
                    acc.at[pl.ds(r0, RPT), :])
    plsc.subcore_barrier()

    def fire(j, r):
      pltpu.sync_copy(dst_hbm.at[pl.ds((start + r) * G, G)], idxs[j])
      pltpu.async_copy(ones_v, acc.at[idxs[j]], sems[j], add=True)

    def drain(j):
      pltpu.make_async_copy(ones_hbm, ones_v, sems[j]).wait()

    for j in range(4):
      @pl.when(j < nrows)
      def _(j=j):
        fire(j, j)

    def body(i, _):
      for j in range(4):
        r = 4 + 4 * i + j
        @pl.when(r < nrows)
        def _(j=j, r=r):
          drain(j)
          fire(j, r)
      return _

    lax.fori_loop(0, (RPW - 4) // 4, body, None)
    for j in range(4):
      @pl.when(j < nrows)
      def _(j=j):
        drain(j)

    plsc.subcore_barrier()
    pltpu.sync_copy(acc.at[pl.ds(r0, RPT), :],
                    deg_hbm.at[c, pl.ds(r0, RPT), :])

  return k


def _make_sc_agg(nb):
  scratch = (
      [pltpu.VMEM((8, G), jnp.int32) for _ in range(4)]
      + [pltpu.VMEM((G, D), _f32) for _ in range(2)]
      + [pltpu.VMEM_SHARED((NP, D), _f32)]
      + [pltpu.SemaphoreType.DMA for _ in range(4)]
  )

  @functools.partial(
      pl.kernel,
      out_type=jax.ShapeDtypeStruct((NC * nb, NP, D), _f32),
      mesh=_sc_mesh(),
      scratch_types=scratch,
  )
  def k(src2_hbm, dst2_hbm, zeros_hbm, *rest):
    g_blocks = rest[:nb]
    z_hbm = rest[nb]
    sb0, sb1, db0, db1, b0, b1, acc, gs0, gs1, ss0, ss1 = rest[nb + 1:]
    sidxb = (sb0, sb1)
    didxb = (db0, db1)
    bufs = (b0, b1)
    gsems = (gs0, gs1)
    ssems = (ss0, ss1)
    c = lax.axis_index("c")
    s = lax.axis_index("s")
    wid = s * NC + c
    start = RPW * wid
    nrows = jnp.clip(NROW - start, 0, RPW)
    rb0 = s * RPT

    def load_idx(p, row0, cond):
      @pl.when(cond)
      def _():
        pltpu.sync_copy(src2_hbm.at[pl.ds(start + row0, 8), :], sidxb[p])
        pltpu.sync_copy(dst2_hbm.at[pl.ds(start + row0, 8), :], didxb[p])

    for cb in range(nb):
      gcb = g_blocks[cb]

      def wait_gather(j, gcb=gcb):
        pltpu.make_async_copy(gcb.at[pl.ds(0, G)], bufs[j], gsems[j]).wait()

      def wait_scatter(j):
        pltpu.make_async_copy(bufs[j], acc.at[pl.ds(0, G)], ssems[j]).wait()

      pltpu.sync_copy(zeros_hbm.at[pl.ds(rb0, RPT), :],
                      acc.at[pl.ds(rb0, RPT), :])
      plsc.subcore_barrier()

      load_idx(0, 0, 0 < nrows)

      def body(bi, _, gcb=gcb):
        for k16 in range(16):
          r = 16 * bi + k16
          j = k16 % 2
          jp = 1 - j
          p = (k16 // 8) % 2
          krow = k16 % 8
          pp = ((k16 - 1) // 8) % 2 if k16 > 0 else 1
          kp = (k16 - 1) % 8

          @pl.when((r >= 2) & (r - 2 < nrows))
          def _(j=j):
            wait_scatter(j)
          @pl.when(r < nrows)
          def _(j=j, p=p, krow=krow):
            pltpu.async_copy(gcb.at[sidxb[p].at[krow]], bufs[j], gsems[j])
          @pl.when((r >= 1) & (r - 1 < nrows))
          def _(jp=jp, pp=pp, kp=kp):
            wait_gather(jp)
            pltpu.async_copy(bufs[jp], acc.at[didxb[pp].at[kp]], ssems[jp],
                             add=True)
          if k16 == 4:
            load_idx(1, 16 * bi + 8, 16 * bi + 8 < nrows)
          if k16 == 12:
            load_idx(0, 16 * bi + 16, 16 * bi + 16 < nrows)
        return _

      lax.fori_loop(0, RPW // 16, body, None)

      @pl.when(nrows >= RPW)
      def _():
        wait_gather(1)
        pltpu.async_copy(bufs[1], acc.at[didxb[1].at[7]], ssems[1], add=True)
        wait_scatter(0)
        wait_scatter(1)

      plsc.subcore_barrier()
      pltpu.sync_copy(acc.at[pl.ds(rb0, RPT), :],
                      z_hbm.at[c * nb + cb, pl.ds(rb0, RPT), :])

  return k


def _tc_prep(deg16, x):
  def body(deg_ref, x_ref, dinv_ref, g0_ref):
    deg = deg_ref[0, :, 0] + deg_ref[1, :, 0] + 1.0
    dinv = 1.0 / lax.sqrt(deg)
    db = jnp.broadcast_to(dinv[:, None], (BR, D))
    dinv_ref[...] = db
    g0_ref[...] = db * x_ref[...]

  return pl.pallas_call(
      body,
      grid=(GRID,),
      in_specs=[
          pl.BlockSpec((NC, BR, D), lambda i: (0, i, 0)),
          pl.BlockSpec((BR, D), lambda i: (i, 0)),
      ],
      out_specs=[
          pl.BlockSpec((BR, D), lambda i: (i, 0)),
          pl.BlockSpec((BR, D), lambda i: (i, 0)),
      ],
      out_shape=[
          jax.ShapeDtypeStruct((N, D), _f32),
          jax.ShapeDtypeStruct((N, D), _f32),
      ],
  )(deg16, x)


def _dot_t(a, w):
  return lax.dot_general(a, w, (((1,), (1,)), ((), ())),
                         precision=lax.Precision.DEFAULT,
                         preferred_element_type=_f32)


def _make_tc_mm(nb, dout):
  nbo = dout // D

  def body(*refs):
    z_ref = refs[0]
    g_refs = refs[1:1 + nb]
    dinv_ref, w_ref, b_ref = refs[1 + nb:4 + nb]
    out_refs = refs[4 + nb:]
    dinv = dinv_ref[...]
    z = z_ref[...]
    s = jnp.concatenate(
        [(z[cb] + z[nb + cb] + g_refs[cb][...]) * dinv for cb in range(nb)],
        axis=1)
    h = jax.nn.relu(_dot_t(s, w_ref[...]) + b_ref[...])
    for ob in range(nbo):
      out_refs[ob][...] = h[:, ob * D:(ob + 1) * D] * dinv

  def run(z, gs, dinv_b, w, b):
    din = nb * D
    return pl.pallas_call(
        body,
        grid=(GRID,),
        in_specs=[pl.BlockSpec((NC * nb, BR, D), lambda i: (0, i, 0))]
        + [pl.BlockSpec((BR, D), lambda i: (i, 0)) for _ in range(nb)]
        + [
            pl.BlockSpec((BR, D), lambda i: (i, 0)),
            pl.BlockSpec((dout, din), lambda i: (0, 0)),
            pl.BlockSpec((1, dout), lambda i: (0, 0)),
        ],
        out_specs=[pl.BlockSpec((BR, D), lambda i: (i, 0))
                   for _ in range(nbo)],
        out_shape=[jax.ShapeDtypeStruct((N, D), _f32) for _ in range(nbo)],
    )(z, *gs, dinv_b, w, b)

  return run


def _tc_head(z3, gs, dinv_b, wc3, bc3, wl1, bl1, wl2, bl2, wl3, bl3):
  nb = 4

  def body(z_ref, g0_ref, g1_ref, g2_ref, g3_ref, dinv_ref,
           wc3_ref, bc3_ref, wl1_ref, bl1_ref, wl2_ref, bl2_ref,
           wl3_ref, bl3_ref, out_ref):
    g_refs = (g0_ref, g1_ref, g2_ref, g3_ref)
    dinv = dinv_ref[...]
    z = z_ref[...]
    s = jnp.concatenate(
        [(z[cb] + z[nb + cb] + g_refs[cb][...]) * dinv for cb in range(nb)],
        axis=1)
    h = jax.nn.relu(_dot_t(s, wc3_ref[...]) + bc3_ref[...])
    h = jax.nn.relu(_dot_t(h, wl1_ref[...]) + bl1_ref[...])
    h = jax.nn.relu(_dot_t(h, wl2_ref[...]) + bl2_ref[...])
    out_ref[...] = jax.nn.relu(_dot_t(h, wl3_ref[...]) + bl3_ref[...])

  def wspec(w):
    return pl.BlockSpec(w.shape, lambda i: (0, 0))

  return pl.pallas_call(
      body,
      grid=(GRID,),
      in_specs=[pl.BlockSpec((NC * nb, BR, D), lambda i: (0, i, 0))]
      + [pl.BlockSpec((BR, D), lambda i: (i, 0)) for _ in range(nb)]
      + [pl.BlockSpec((BR, D), lambda i: (i, 0))]
      + [wspec(wc3), wspec(bc3), wspec(wl1), wspec(bl1),
         wspec(wl2), wspec(bl2), wspec(wl3), wspec(bl3)],
      out_specs=pl.BlockSpec((BR, D), lambda i: (i, 0)),
      out_shape=jax.ShapeDtypeStruct((N, D), _f32),
  )(z3, *gs, dinv_b, wc3, bc3, wl1, bl1, wl2, bl2, wl3, bl3)


_sc_deg = _make_sc_deg()
_sc_agg1 = _make_sc_agg(1)
_sc_agg2 = _make_sc_agg(2)
_sc_agg3 = _make_sc_agg(4)
_tc_mm1 = _make_tc_mm(1, 2 * D)
_tc_mm2 = _make_tc_mm(2, 4 * D)


def kernel(x, edge_index, Wc1, bc1, Wc2, bc2, Wc3, bc3,
           Wl1, bl1, Wl2, bl2, Wl3, bl3):
  src1d = edge_index[0]
  ei_pad = jnp.concatenate(
      [edge_index, jnp.zeros((2, NROWP * G - E), jnp.int32)], axis=1)
  src2d = ei_pad[0].reshape(NROWP, G)
  dst2d = ei_pad[1].reshape(NROWP, G)
  zeros_hbm = jnp.zeros((NP, D), _f32)
  ones128 = jnp.ones((G, D), _f32)

  deg16 = _sc_deg(edge_index[1], zeros_hbm, ones128)
  dinv_b, g0 = _tc_prep(deg16, x)

  z1 = _sc_agg1(src2d, dst2d, zeros_hbm, g0)
  g1 = _tc_mm1(z1, (g0,), dinv_b, Wc1, bc1.reshape(1, -1))

  z2 = _sc_agg2(src2d, dst2d, zeros_hbm, *g1)
  g2 = _tc_mm2(z2, g1, dinv_b, Wc2, bc2.reshape(1, -1))

  z3 = _sc_agg3(src2d, dst2d, zeros_hbm, *g2)
  out = _tc_head(z3, g2, dinv_b, Wc3, bc3.reshape(1, -1),
                 Wl1, bl1.reshape(1, -1), Wl2, bl2.reshape(1, -1),
                 Wl3, bl3.reshape(1, -1))
  return out


# speedup vs baseline: 20.4979x; 1.0145x over previous
import functools

import jax
import jax.numpy as jnp
from jax import lax
from jax.experimental import pallas as pl
from jax.experimental.pallas import tpu as pltpu
from jax.experimental.pallas import tpu_sc as plsc

N = 10000
E = 320000
D = 128

NC = 2
NS = 16
NW = NC * NS
G = 128
NROW = E // G
RPW = 80
NROWP = 2504
NP = 10112
RPT = NP // NS
BR = 400
GRID = N // BR

_f32 = jnp.float32


def _sc_mesh():
  return plsc.VectorSubcoreMesh(core_axis_name="c", subcore_axis_name="s")


def _make_sc_deg():
  scratch = [
      pltpu.VMEM((G, D), _f32),
      pltpu.VMEM((8, G), jnp.int32),
      pltpu.VMEM((8, G), jnp.int32),
      pltpu.VMEM_SHARED((NP, D), _f32),
  ] + [pltpu.SemaphoreType.DMA for _ in range(4)]

  @functools.partial(
      pl.kernel,
      out_type=jax.ShapeDtypeStruct((NC, NP, D), _f32),
      mesh=_sc_mesh(),
      scratch_types=scratch,
  )
  def k(dst2_hbm, zeros_hbm, ones_hbm, deg_hbm, ones_v, db0, db1, acc,
        s0, s1, s2, s3):
    didxb = (db0, db1)
    sems = (s0, s1, s2, s3)
    c = lax.axis_index("c")
    s = lax.axis_index("s")
    wid = s * NC + c
    start = RPW * wid
    nrows = jnp.clip(NROW - start, 0, RPW)
    r0 = s * RPT

    def load_didx(p, row0, cond):
      @pl.when(cond)
      def _():
        pltpu.sync_copy(dst2_hbm.at[pl.ds(start + row0, 8), :], didxb[p])

    def drain(j):
      pltpu.make_async_copy(ones_hbm, ones_v, sems[j]).wait()

    pltpu.sync_copy(ones_hbm, ones_v)
    pltpu.sync_copy(zeros_hbm.at[pl.ds(r0, RPT), :],
                    acc.at[pl.ds(r0, RPT), :])
    plsc.subcore_barrier()

    load_didx(0, 0, 0 < nrows)

    def body(bi, _):
      for k16 in range(16):
        r = 16 * bi + k16
        j = k16 % 4
        p = (k16 // 8) % 2
        krow = k16 % 8
        @pl.when((r >= 4) & (r - 4 < nrows))
        def _(j=j):
          drain(j)
        @pl.when(r < nrows)
        def _(j=j, p=p, krow=krow):
          pltpu.async_copy(ones_v, acc.at[didxb[p].at[krow]], sems[j],
                           add=True)
        if k16 == 4:
          load_didx(1, 16 * bi + 8, 16 * bi + 8 < nrows)
        if k16 == 12:
          load_didx(0, 16 * bi + 16, 16 * bi + 16 < nrows)
      return _

    lax.fori_loop(0, RPW // 16, body, None)
    for j in range(4):
      @pl.when(nrows >= RPW)
      def _(j=j):
        drain(j)

    plsc.subcore_barrier()
    pltpu.sync_copy(acc.at[pl.ds(r0, RPT), :],
                    deg_hbm.at[c, pl.ds(r0, RPT), :])

  return k


def _make_sc_agg(nb):
  scratch = (
      [pltpu.VMEM((8, G), jnp.int32) for _ in range(4)]
      + [pltpu.VMEM((G, D), _f32) for _ in range(2)]
      + [pltpu.VMEM_SHARED((NP, D), _f32)]
      + [pltpu.SemaphoreType.DMA for _ in range(4)]
  )

  @functools.partial(
      pl.kernel,
      out_type=jax.ShapeDtypeStruct((NC * nb, NP, D), _f32),
      mesh=_sc_mesh(),
      scratch_types=scratch,
  )
  def k(src2_hbm, dst2_hbm, zeros_hbm, *rest):
    g_blocks = rest[:nb]
    z_hbm = rest[nb]
    sb0, sb1, db0, db1, b0, b1, acc, gs0, gs1, ss0, ss1 = rest[nb + 1:]
    sidxb = (sb0, sb1)
    didxb = (db0, db1)
    bufs = (b0, b1)
    gsems = (gs0, gs1)
    ssems = (ss0, ss1)
    c = lax.axis_index("c")
    s = lax.axis_index("s")
    wid = s * NC + c
    start = RPW * wid
    nrows = jnp.clip(NROW - start, 0, RPW)
    rb0 = s * RPT

    def load_idx(p, row0, cond):
      @pl.when(cond)
      def _():
        pltpu.sync_copy(src2_hbm.at[pl.ds(start + row0, 8), :], sidxb[p])
        pltpu.sync_copy(dst2_hbm.at[pl.ds(start + row0, 8), :], didxb[p])

    for cb in range(nb):
      gcb = g_blocks[cb]

      def wait_gather(j, gcb=gcb):
        pltpu.make_async_copy(gcb.at[pl.ds(0, G)], bufs[j], gsems[j]).wait()

      def wait_scatter(j):
        pltpu.make_async_copy(bufs[j], acc.at[pl.ds(0, G)], ssems[j]).wait()

      pltpu.sync_copy(zeros_hbm.at[pl.ds(rb0, RPT), :],
                      acc.at[pl.ds(rb0, RPT), :])
      plsc.subcore_barrier()

      load_idx(0, 0, 0 < nrows)

      def body(bi, _, gcb=gcb):
        for k16 in range(16):
          r = 16 * bi + k16
          j = k16 % 2
          jp = 1 - j
          p = (k16 // 8) % 2
          krow = k16 % 8
          pp = ((k16 - 1) // 8) % 2 if k16 > 0 else 1
          kp = (k16 - 1) % 8

          @pl.when((r >= 2) & (r - 2 < nrows))
          def _(j=j):
            wait_scatter(j)
          @pl.when(r < nrows)
          def _(j=j, p=p, krow=krow):
            pltpu.async_copy(gcb.at[sidxb[p].at[krow]], bufs[j], gsems[j])
          @pl.when((r >= 1) & (r - 1 < nrows))
          def _(jp=jp, pp=pp, kp=kp):
            wait_gather(jp)
            pltpu.async_copy(bufs[jp], acc.at[didxb[pp].at[kp]], ssems[jp],
                             add=True)
          if k16 == 4:
            load_idx(1, 16 * bi + 8, 16 * bi + 8 < nrows)
          if k16 == 12:
            load_idx(0, 16 * bi + 16, 16 * bi + 16 < nrows)
        return _

      lax.fori_loop(0, RPW // 16, body, None)

      @pl.when(nrows >= RPW)
      def _():
        wait_gather(1)
        pltpu.async_copy(bufs[1], acc.at[didxb[1].at[7]], ssems[1], add=True)
        wait_scatter(0)
        wait_scatter(1)

      plsc.subcore_barrier()
      pltpu.sync_copy(acc.at[pl.ds(rb0, RPT), :],
                      z_hbm.at[c * nb + cb, pl.ds(rb0, RPT), :])

  return k


def _tc_prep(deg16, x):
  def body(deg_ref, x_ref, dinv_ref, g0_ref):
    deg = deg_ref[0, :, 0] + deg_ref[1, :, 0] + 1.0
    dinv = 1.0 / lax.sqrt(deg)
    db = jnp.broadcast_to(dinv[:, None], (BR, D))
    dinv_ref[...] = db
    g0_ref[...] = db * x_ref[...]

  return pl.pallas_call(
      body,
      grid=(GRID,),
      in_specs=[
          pl.BlockSpec((NC, BR, D), lambda i: (0, i, 0)),
          pl.BlockSpec((BR, D), lambda i: (i, 0)),
      ],
      out_specs=[
          pl.BlockSpec((BR, D), lambda i: (i, 0)),
          pl.BlockSpec((BR, D), lambda i: (i, 0)),
      ],
      out_shape=[
          jax.ShapeDtypeStruct((N, D), _f32),
          jax.ShapeDtypeStruct((N, D), _f32),
      ],
  )(deg16, x)


def _dot_t(a, w):
  return lax.dot_general(a, w, (((1,), (1,)), ((), ())),
                         precision=lax.Precision.DEFAULT,
                         preferred_element_type=_f32)


def _make_tc_mm(nb, dout):
  nbo = dout // D

  def body(*refs):
    z_ref = refs[0]
    g_refs = refs[1:1 + nb]
    dinv_ref, w_ref, b_ref = refs[1 + nb:4 + nb]
    out_refs = refs[4 + nb:]
    dinv = dinv_ref[...]
    z = z_ref[...]
    s = jnp.concatenate(
        [(z[cb] + z[nb + cb] + g_refs[cb][...]) * dinv for cb in range(nb)],
        axis=1)
    h = jax.nn.relu(_dot_t(s, w_ref[...]) + b_ref[...])
    for ob in range(nbo):
      out_refs[ob][...] = h[:, ob * D:(ob + 1) * D] * dinv

  def run(z, gs, dinv_b, w, b):
    din = nb * D
    return pl.pallas_call(
        body,
        grid=(GRID,),
        in_specs=[pl.BlockSpec((NC * nb, BR, D), lambda i: (0, i, 0))]
        + [pl.BlockSpec((BR, D), lambda i: (i, 0)) for _ in range(nb)]
        + [
            pl.BlockSpec((BR, D), lambda i: (i, 0)),
            pl.BlockSpec((dout, din), lambda i: (0, 0)),
            pl.BlockSpec((1, dout), lambda i: (0, 0)),
        ],
        out_specs=[pl.BlockSpec((BR, D), lambda i: (i, 0))
                   for _ in range(nbo)],
        out_shape=[jax.ShapeDtypeStruct((N, D), _f32) for _ in range(nbo)],
    )(z, *gs, dinv_b, w, b)

  return run


def _tc_head(z3, gs, dinv_b, wc3, bc3, wl1, bl1, wl2, bl2, wl3, bl3):
  nb = 4

  def body(z_ref, g0_ref, g1_ref, g2_ref, g3_ref, dinv_ref,
           wc3_ref, bc3_ref, wl1_ref, bl1_ref, wl2_ref, bl2_ref,
           wl3_ref, bl3_ref, out_ref):
    g_refs = (g0_ref, g1_ref, g2_ref, g3_ref)
    dinv = dinv_ref[...]
    z = z_ref[...]
    s = jnp.concatenate(
        [(z[cb] + z[nb + cb] + g_refs[cb][...]) * dinv for cb in range(nb)],
        axis=1)
    h = jax.nn.relu(_dot_t(s, wc3_ref[...]) + bc3_ref[...])
    h = jax.nn.relu(_dot_t(h, wl1_ref[...]) + bl1_ref[...])
    h = jax.nn.relu(_dot_t(h, wl2_ref[...]) + bl2_ref[...])
    out_ref[...] = jax.nn.relu(_dot_t(h, wl3_ref[...]) + bl3_ref[...])

  def wspec(w):
    return pl.BlockSpec(w.shape, lambda i: (0, 0))

  return pl.pallas_call(
      body,
      grid=(GRID,),
      in_specs=[pl.BlockSpec((NC * nb, BR, D), lambda i: (0, i, 0))]
      + [pl.BlockSpec((BR, D), lambda i: (i, 0)) for _ in range(nb)]
      + [pl.BlockSpec((BR, D), lambda i: (i, 0))]
      + [wspec(wc3), wspec(bc3), wspec(wl1), wspec(bl1),
         wspec(wl2), wspec(bl2), wspec(wl3), wspec(bl3)],
      out_specs=pl.BlockSpec((BR, D), lambda i: (i, 0)),
      out_shape=jax.ShapeDtypeStruct((N, D), _f32),
  )(z3, *gs, dinv_b, wc3, bc3, wl1, bl1, wl2, bl2, wl3, bl3)


_sc_deg = _make_sc_deg()
_sc_agg1 = _make_sc_agg(1)
_sc_agg2 = _make_sc_agg(2)
_sc_agg3 = _make_sc_agg(4)
_tc_mm1 = _make_tc_mm(1, 2 * D)
_tc_mm2 = _make_tc_mm(2, 4 * D)


def kernel(x, edge_index, Wc1, bc1, Wc2, bc2, Wc3, bc3,
           Wl1, bl1, Wl2, bl2, Wl3, bl3):
  ei_pad = jnp.concatenate(
      [edge_index, jnp.zeros((2, NROWP * G - E), jnp.int32)], axis=1)
  src2d = ei_pad[0].reshape(NROWP, G)
  dst2d = ei_pad[1].reshape(NROWP, G)
  zeros_hbm = jnp.zeros((NP, D), _f32)
  ones128 = jnp.ones((G, D), _f32)

  deg16 = _sc_deg(dst2d, zeros_hbm, ones128)
  dinv_b, g0 = _tc_prep(deg16, x)

  z1 = _sc_agg1(src2d, dst2d, zeros_hbm, g0)
  g1 = _tc_mm1(z1, (g0,), dinv_b, Wc1, bc1.reshape(1, -1))

  z2 = _sc_agg2(src2d, dst2d, zeros_hbm, *g1)
  g2 = _tc_mm2(z2, g1, dinv_b, Wc2, bc2.reshape(1, -1))

  z3 = _sc_agg3(src2d, dst2d, zeros_hbm, *g2)
  out = _tc_head(z3, g2, dinv_b, Wc3, bc3.reshape(1, -1),
                 Wl1, bl1.reshape(1, -1), Wl2, bl2.reshape(1, -1),
                 Wl3, bl3.reshape(1, -1))
  return out
